# Initial kernel scaffold; baseline (speedup 1.0000x reference)
import jax
import jax.numpy as jnp
from jax.experimental import pallas as pl


def kernel(x, edge_index, W_neigh, b_neigh):
    raise NotImplementedError("write your pallas kernel here")


# speedup vs baseline: 11.0579x; 11.0579x over previous
import functools

import jax
import jax.numpy as jnp
from jax import lax
from jax.experimental import pallas as pl
from jax.experimental.pallas import tpu as pltpu
from jax.experimental.pallas import tpu_sc as plsc

N_NODES = 10000
N_EDGES = 320000
D = 128
DH = 64
DP = 80

NC = 2
NS = 16
EDGES_PER_TILE = N_EDGES // NS
CHUNK = 125
NCHUNK = EDGES_PER_TILE // CHUNK
ROWS_PER_TILE = N_NODES // NS

_sc_mesh = plsc.VectorSubcoreMesh(
    core_axis_name="c", subcore_axis_name="s", num_cores=NC, num_subcores=NS
)


@functools.partial(
    pl.kernel,
    out_type=jax.ShapeDtypeStruct((NC, N_NODES, DP), jnp.float32),
    mesh=_sc_mesh,
    scratch_types=[
        pltpu.VMEM((NCHUNK, CHUNK), jnp.int32),
        pltpu.VMEM((NCHUNK, CHUNK), jnp.int32),
        pltpu.VMEM((2, CHUNK, DP), jnp.float32),
        pltpu.VMEM_SHARED((N_NODES, DP), jnp.float32),
        pltpu.SemaphoreType.DMA,
        pltpu.SemaphoreType.DMA,
    ],
    compiler_params=pltpu.CompilerParams(use_tc_tiling_on_sc=False),
)
def _aggregate(xh_hbm, edges_hbm, zeros_hbm, out_hbm,
               src_v, dst_v, rows_v, acc_sh, sem0, sem1):
    cid = lax.axis_index("c")
    sid = lax.axis_index("s")
    xp = xh_hbm.at[cid]

    pltpu.sync_copy(edges_hbm.at[0, sid], src_v)
    pltpu.sync_copy(edges_hbm.at[1, sid], dst_v)

    pltpu.sync_copy(zeros_hbm, acc_sh.at[pl.ds(sid * ROWS_PER_TILE, ROWS_PER_TILE)])
    plsc.subcore_barrier()

    pltpu.async_copy(xp.at[src_v.at[0]], rows_v.at[0], sem0)

    def body(jj, _):
        j0 = 2 * jj
        pltpu.async_copy(xp.at[src_v.at[j0 + 1]], rows_v.at[1], sem1)
        pltpu.make_async_copy(xp.at[src_v.at[j0]], rows_v.at[0], sem0).wait()
        pltpu.sync_copy(rows_v.at[0], acc_sh.at[dst_v.at[j0]], add=True)

        @pl.when(j0 + 2 < NCHUNK)
        def _():
            pltpu.async_copy(xp.at[src_v.at[j0 + 2]], rows_v.at[0], sem0)

        pltpu.make_async_copy(xp.at[src_v.at[j0 + 1]], rows_v.at[1], sem1).wait()
        pltpu.sync_copy(rows_v.at[1], acc_sh.at[dst_v.at[j0 + 1]], add=True)
        return 0

    lax.fori_loop(0, NCHUNK // 2, body, 0)

    plsc.subcore_barrier()
    row0 = sid * ROWS_PER_TILE
    pltpu.sync_copy(
        acc_sh.at[pl.ds(row0, ROWS_PER_TILE)],
        out_hbm.at[cid, pl.ds(row0, ROWS_PER_TILE)],
    )


ROW_BLK = 2000


def _finish_body(parts_ref, wt0_ref, wt1_ref, b_ref, out_ref):
    lo = parts_ref[0]
    hi = parts_ref[1]
    inv = 1.0 / jnp.maximum(lo[:, DH:DH + 1], 1.0)
    h0 = lo[:, :DH] * inv
    h1 = hi[:, :DH] * inv
    out_ref[...] = (
        jnp.dot(h0, wt0_ref[...], preferred_element_type=jnp.float32)
        + jnp.dot(h1, wt1_ref[...], preferred_element_type=jnp.float32)
        + b_ref[...] + 0.01
    )


_finish = pl.pallas_call(
    _finish_body,
    grid=(N_NODES // ROW_BLK,),
    in_specs=[
        pl.BlockSpec((NC, ROW_BLK, DP), lambda i: (0, i, 0)),
        pl.BlockSpec((DH, D), lambda i: (0, 0)),
        pl.BlockSpec((DH, D), lambda i: (0, 0)),
        pl.BlockSpec((1, D), lambda i: (0, 0)),
    ],
    out_specs=pl.BlockSpec((ROW_BLK, D), lambda i: (i, 0)),
    out_shape=jax.ShapeDtypeStruct((N_NODES, D), jnp.float32),
)


@jax.jit
def kernel(x, edge_index, W_neigh, b_neigh):
    ones = jnp.ones((N_NODES, 1), x.dtype)
    pad = jnp.zeros((N_NODES, DP - DH - 1), x.dtype)
    xh = jnp.stack(
        [
            jnp.concatenate([x[:, :DH], ones, pad], axis=1),
            jnp.concatenate([x[:, DH:], ones, pad], axis=1),
        ]
    )
    edges = edge_index.reshape(2, NS, NCHUNK, CHUNK)
    zeros = jnp.zeros((ROWS_PER_TILE, DP), jnp.float32)
    parts = _aggregate(xh, edges, zeros)
    wt = W_neigh.T
    return _finish(parts, wt[:DH], wt[DH:], b_neigh.reshape(1, D))
